# baseline (speedup 1.0000x reference)
import functools

import jax
import jax.numpy as jnp
from jax import lax
from jax.experimental import pallas as pl
from jax.experimental.pallas import tpu as pltpu
from jax.experimental.pallas import tpu_sc as plsc

NUM_EMB = 1_000_000
D = 64
B = 4096 * 200
NW = 32
B_PER_W = B // NW
CHUNK = 1600
NCHUNK = B_PER_W // CHUNK


@functools.partial(
    pl.kernel,
    mesh=plsc.VectorSubcoreMesh(core_axis_name="c", subcore_axis_name="s"),
    out_type=jax.ShapeDtypeStruct((B, D), jnp.float32),
    compiler_params=pltpu.CompilerParams(use_tc_tiling_on_sc=False),
    scratch_types=[
        pltpu.VMEM((CHUNK,), jnp.int32),
        pltpu.VMEM((CHUNK, D), jnp.float32),
        pltpu.SemaphoreType.DMA,
    ],
)
def _gather_kernel(item_hbm, table_hbm, out_hbm, idx_v, rows_v, sem):
    wid = lax.axis_index("s") * 2 + lax.axis_index("c")
    base = wid * B_PER_W

    def chunk_body(c, carry):
        off = base + c * CHUNK
        pltpu.sync_copy(item_hbm.at[pl.ds(off, CHUNK)], idx_v)
        pltpu.async_copy(table_hbm.at[idx_v], rows_v, sem).wait()
        pltpu.sync_copy(rows_v, out_hbm.at[pl.ds(off, CHUNK)])
        return carry

    lax.fori_loop(0, NCHUNK, chunk_body, 0)


def kernel(item, table):
    flat = item.reshape((B,)).astype(jnp.int32)
    out = _gather_kernel(flat, table)
    return out.reshape(item.shape + (D,))


# speedup vs baseline: 1.0078x; 1.0078x over previous
import functools

import jax
import jax.numpy as jnp
from jax import lax
from jax.experimental import pallas as pl
from jax.experimental.pallas import tpu as pltpu
from jax.experimental.pallas import tpu_sc as plsc

D = 64
B = 4096 * 200
NW = 32
B_PER_W = B // NW
C = 800
NB = 2
NCH = B_PER_W // C
NG = NCH // NB


@functools.partial(
    pl.kernel,
    mesh=plsc.VectorSubcoreMesh(core_axis_name="c", subcore_axis_name="s"),
    out_type=jax.ShapeDtypeStruct((B, D), jnp.float32),
    compiler_params=pltpu.CompilerParams(use_tc_tiling_on_sc=False),
    scratch_types=[
        pltpu.VMEM((C,), jnp.int32),
        pltpu.VMEM((C,), jnp.int32),
        pltpu.VMEM((C, D), jnp.float32),
        pltpu.VMEM((C, D), jnp.float32),
        pltpu.SemaphoreType.DMA,
        pltpu.SemaphoreType.DMA,
        pltpu.SemaphoreType.DMA,
        pltpu.SemaphoreType.DMA,
        pltpu.SemaphoreType.DMA,
        pltpu.SemaphoreType.DMA,
    ],
)
def _gather_kernel(item_hbm, table_hbm, out_hbm,
                   idx0, idx1, rows0, rows1,
                   si0, si1, sg0, sg1, so0, so1):
    wid = lax.axis_index("s") * 2 + lax.axis_index("c")
    base = wid * B_PER_W
    idxs = (idx0, idx1)
    rows = (rows0, rows1)
    sis = (si0, si1)
    sgs = (sg0, sg1)
    sos = (so0, so1)

    def step(off, b, wait_prev_out, prefetch_next):
        pltpu.make_async_copy(
            item_hbm.at[pl.ds(off, C)], idxs[b], sis[b]).wait()
        if wait_prev_out:
            pltpu.make_async_copy(
                rows[b], out_hbm.at[pl.ds(off - NB * C, C)], sos[b]).wait()
        pltpu.async_copy(table_hbm.at[idxs[b]], rows[b], sgs[b]).wait()
        if prefetch_next:
            pltpu.async_copy(
                item_hbm.at[pl.ds(off + NB * C, C)], idxs[b], sis[b])
        pltpu.async_copy(rows[b], out_hbm.at[pl.ds(off, C)], sos[b])

    for b in range(NB):
        pltpu.async_copy(item_hbm.at[pl.ds(base + b * C, C)], idxs[b], sis[b])

    for b in range(NB):
        step(base + b * C, b, wait_prev_out=False, prefetch_next=True)

    def outer(g, carry):
        for b in range(NB):
            step(base + (g * NB + b) * C, b,
                 wait_prev_out=True, prefetch_next=True)
        return carry

    lax.fori_loop(1, NG - 1, outer, 0)

    for b in range(NB):
        step(base + ((NG - 1) * NB + b) * C, b,
             wait_prev_out=True, prefetch_next=False)

    for b in range(NB):
        off = base + (NCH - NB + b) * C
        pltpu.make_async_copy(rows[b], out_hbm.at[pl.ds(off, C)], sos[b]).wait()


def kernel(item, table):
    flat = item.reshape((B,)).astype(jnp.int32)
    out = _gather_kernel(flat, table)
    return out.reshape(item.shape + (D,))


# speedup vs baseline: 1.0336x; 1.0256x over previous
import functools

import jax
import jax.numpy as jnp
from jax import lax
from jax.experimental import pallas as pl
from jax.experimental.pallas import tpu as pltpu
from jax.experimental.pallas import tpu_sc as plsc

D = 64
B = 4096 * 200
NW = 32
B_PER_W = B // NW
C = 800
NB = 2
NCH = B_PER_W // C
NG = NCH // NB


@functools.partial(
    pl.kernel,
    mesh=plsc.VectorSubcoreMesh(core_axis_name="c", subcore_axis_name="s"),
    out_type=jax.ShapeDtypeStruct((B, D), jnp.float32),
    compiler_params=pltpu.CompilerParams(use_tc_tiling_on_sc=False),
    scratch_types=[
        pltpu.VMEM((C,), jnp.int32),
        pltpu.VMEM((C,), jnp.int32),
        pltpu.VMEM((C, D), jnp.float32),
        pltpu.VMEM((C, D), jnp.float32),
        pltpu.SemaphoreType.DMA,
        pltpu.SemaphoreType.DMA,
        pltpu.SemaphoreType.DMA,
        pltpu.SemaphoreType.DMA,
        pltpu.SemaphoreType.DMA,
        pltpu.SemaphoreType.DMA,
    ],
)
def _gather_kernel(item_hbm, table_hbm, out_hbm,
                   idx0, idx1, rows0, rows1,
                   si0, si1, sg0, sg1, so0, so1):
    wid = lax.axis_index("s") * 2 + lax.axis_index("c")
    base = wid * B_PER_W
    idxs = (idx0, idx1)
    rows = (rows0, rows1)
    sis = (si0, si1)
    sgs = (sg0, sg1)
    sos = (so0, so1)

    def step(off, b, wait_prev_out, prefetch_next):
        pltpu.make_async_copy(
            item_hbm.at[pl.ds(off, C)], idxs[b], sis[b]).wait()
        if wait_prev_out:
            pltpu.make_async_copy(
                rows[b], out_hbm.at[pl.ds(off - NB * C, C)], sos[b]).wait()
        pltpu.async_copy(table_hbm.at[idxs[b]], rows[b], sgs[b]).wait()
        if prefetch_next:
            pltpu.async_copy(
                item_hbm.at[pl.ds(off + NB * C, C)], idxs[b], sis[b])
        pltpu.async_copy(rows[b], out_hbm.at[pl.ds(off, C)], sos[b])

    for b in range(NB):
        pltpu.async_copy(item_hbm.at[pl.ds(base + b * C, C)], idxs[b], sis[b])

    for b in range(NB):
        step(base + b * C, b, wait_prev_out=False, prefetch_next=True)

    def outer(g, carry):
        for b in range(NB):
            step(base + (g * NB + b) * C, b,
                 wait_prev_out=True, prefetch_next=True)
        return carry

    lax.fori_loop(1, NG - 1, outer, 0)

    for b in range(NB):
        step(base + ((NG - 1) * NB + b) * C, b,
             wait_prev_out=True, prefetch_next=False)

    for b in range(NB):
        off = base + (NCH - NB + b) * C
        pltpu.make_async_copy(rows[b], out_hbm.at[pl.ds(off, C)], sos[b]).wait()


def kernel(item, table):
    n_i, n_j = item.shape
    flat = item.T.reshape((B,)).astype(jnp.int32)
    out = _gather_kernel(flat, table)
    return out.reshape((n_j, n_i, D)).transpose((1, 0, 2))
